# baseline (speedup 1.0000x reference)
import jax
import jax.numpy as jnp
from jax import lax
from jax.experimental import pallas as pl
from jax.experimental.pallas import tpu as pltpu
from jax.experimental.pallas import tpu_sc as plsc

N = 10000
E = 160000
D_IN = 256
D_HID = 128
D_OUT = 256

NP = 10240
B = 128
TKA = 64
TKB = 16
TK = TKA + TKB
EP = 16 * TK * B
RPT = NP // 16


def _make_seg_sum(with_deg: bool):
  mesh = plsc.VectorSubcoreMesh(core_axis_name="c", subcore_axis_name="s")
  out_type = [jax.ShapeDtypeStruct((NP, D_HID), jnp.float32)]
  if with_deg:
    out_type.append(jax.ShapeDtypeStruct((NP,), jnp.float32))
  scratch = [
      pltpu.VMEM((4, 2, B), jnp.int32),
      pltpu.VMEM((B, D_HID), jnp.float32),
      pltpu.VMEM((B, D_HID), jnp.float32),
      pltpu.VMEM((B,), jnp.float32),
      pltpu.VMEM((B,), jnp.float32),
      pltpu.VMEM_SHARED((NP, D_HID), jnp.float32),
  ]
  if with_deg:
    scratch.append(pltpu.VMEM_SHARED((NP,), jnp.float32))
  scratch += [pltpu.SemaphoreType.DMA] * 6

  def body(y_hbm, idxa_hbm, idxb_hbm, *rest):
    if with_deg:
      (out_rows, out_deg, idx_v, rows_v0, rows_v1, ones_v, zeros_v,
       acc_sh, deg_sh, si0, si1, si2, si3, sem0, sem1) = rest
    else:
      (out_rows, idx_v, rows_v0, rows_v1, ones_v, zeros_v,
       acc_sh, si0, si1, si2, si3, sem0, sem1) = rest
      out_deg = None
      deg_sh = None
    sis = (si0, si1, si2, si3)
    cid = lax.axis_index("c")
    sid = lax.axis_index("s")
    base = sid * RPT

    one16 = jnp.ones((16,), jnp.float32)
    zero16 = jnp.zeros((16,), jnp.float32)
    for jj in range(B // 16):
      ones_v[pl.ds(jj * 16, 16)] = one16
      zeros_v[pl.ds(jj * 16, 16)] = zero16

    def zero_acc():
      rows_v = rows_v0

      def zero_rows(i, carry):
        for jj in range(D_HID // 16):
          rows_v[i, pl.ds(jj * 16, 16)] = zero16
        return carry
      lax.fori_loop(0, B, zero_rows, 0)
      for bb in range(RPT // B):
        pltpu.sync_copy(rows_v, acc_sh.at[pl.ds(base + bb * B, B)])

    def row_pipeline(idx_hbm, tk):
      def fetch_idx(j, slot):
        pltpu.async_copy(idx_hbm.at[sid, j], idx_v.at[slot], sis[slot])

      for q in range(min(4, tk)):
        fetch_idx(q, q)
      pltpu.make_async_copy(idx_hbm.at[sid, 0], idx_v.at[0], si0).wait()
      pltpu.async_copy(y_hbm.at[idx_v.at[0, 0]], rows_v0, sem0)
      pltpu.make_async_copy(idx_hbm.at[sid, 1], idx_v.at[1], si1).wait()
      pltpu.async_copy(y_hbm.at[idx_v.at[1, 0]], rows_v1, sem1)

      def halfstep(j, slot, buf, sem):
        nslot = (slot + 2) % 4
        pltpu.make_async_copy(y_hbm.at[idx_v.at[slot, 0]], buf, sem).wait()
        pltpu.sync_copy(buf, acc_sh.at[idx_v.at[slot, 1]], add=True)

        @pl.when(j + 4 < tk)
        def _():
          fetch_idx(j + 4, slot)

        @pl.when(j + 2 < tk)
        def _():
          pltpu.make_async_copy(idx_hbm.at[sid, j + 2], idx_v.at[nslot],
                                sis[nslot]).wait()
          pltpu.async_copy(y_hbm.at[idx_v.at[nslot, 0]], buf, sem)

      def step(tt, carry):
        j = 4 * tt
        halfstep(j, 0, rows_v0, sem0)
        halfstep(j + 1, 1, rows_v1, sem1)
        halfstep(j + 2, 2, rows_v0, sem0)
        halfstep(j + 3, 3, rows_v1, sem1)
        return carry
      lax.fori_loop(0, tk // 4, step, 0)

    def deg_pipeline(idx_hbm, tk):
      def fetch_idx(j, slot):
        pltpu.async_copy(idx_hbm.at[sid, j], idx_v.at[slot], sis[slot])

      for q in range(min(4, tk)):
        fetch_idx(q, q)

      def dhalf(j, slot):
        pltpu.make_async_copy(idx_hbm.at[sid, j], idx_v.at[slot],
                              sis[slot]).wait()
        pltpu.sync_copy(ones_v, deg_sh.at[idx_v.at[slot, 1]], add=True)

        @pl.when(j + 4 < tk)
        def _():
          fetch_idx(j + 4, slot)

      def dstep(tt, carry):
        j = 4 * tt
        dhalf(j, 0)
        dhalf(j + 1, 1)
        dhalf(j + 2, 2)
        dhalf(j + 3, 3)
        return carry
      lax.fori_loop(0, tk // 4, dstep, 0)

    @pl.when(cid == 0)
    def _():
      zero_acc()
      plsc.subcore_barrier()
      row_pipeline(idxa_hbm, TKA)
      row_pipeline(idxb_hbm, TKB)
      plsc.subcore_barrier()
      pltpu.sync_copy(acc_sh.at[pl.ds(base, RPT)],
                      out_rows.at[pl.ds(base, RPT)])

    if with_deg:
      @pl.when(cid == 1)
      def _():
        for bb in range(RPT // B):
          pltpu.sync_copy(zeros_v, deg_sh.at[pl.ds(base + bb * B, B)])
        plsc.subcore_barrier()
        deg_pipeline(idxa_hbm, TKA)
        deg_pipeline(idxb_hbm, TKB)
        plsc.subcore_barrier()
        pltpu.sync_copy(deg_sh.at[pl.ds(base, RPT)],
                        out_deg.at[pl.ds(base, RPT)])

  return pl.kernel(body, mesh=mesh, out_type=out_type, scratch_types=scratch)


_seg_sum_deg = _make_seg_sum(with_deg=True)
_seg_sum = _make_seg_sum(with_deg=False)


def _proj_body(x_ref, wl_ref, wr_ref, b1_ref, y_ref, z_ref):
  x = x_ref[...]
  y_ref[...] = jnp.dot(x, wl_ref[...], preferred_element_type=jnp.float32)
  z_ref[...] = (jnp.dot(x, wr_ref[...], preferred_element_type=jnp.float32)
                + b1_ref[...])


def _layer1_proj(x, W1_l, W1_r, b1):
  blk = 400
  grid = N // blk
  return pl.pallas_call(
      _proj_body,
      grid=(grid,),
      in_specs=[
          pl.BlockSpec((blk, D_IN), lambda i: (i, 0)),
          pl.BlockSpec((D_IN, D_HID), lambda i: (0, 0)),
          pl.BlockSpec((D_IN, D_HID), lambda i: (0, 0)),
          pl.BlockSpec((1, D_HID), lambda i: (0, 0)),
      ],
      out_specs=[
          pl.BlockSpec((blk, D_HID), lambda i: (i, 0)),
          pl.BlockSpec((blk, D_HID), lambda i: (i, 0)),
      ],
      out_shape=[
          jax.ShapeDtypeStruct((N, D_HID), jnp.float32),
          jax.ShapeDtypeStruct((N, D_HID), jnp.float32),
      ],
  )(x, W1_l, W1_r, b1.reshape(1, D_HID))


def _combine1_body(p_ref, dg_ref, z_ref, h_ref):
  inv = 1.0 / jnp.maximum(dg_ref[...], 1.0)
  h_ref[...] = jnp.maximum(p_ref[...] * inv + z_ref[...], 0.0)


def _combine1(p, deg_col, z):
  blk = 400
  grid = N // blk
  return pl.pallas_call(
      _combine1_body,
      grid=(grid,),
      in_specs=[
          pl.BlockSpec((blk, D_HID), lambda i: (i, 0)),
          pl.BlockSpec((blk, 1), lambda i: (i, 0)),
          pl.BlockSpec((blk, D_HID), lambda i: (i, 0)),
      ],
      out_specs=pl.BlockSpec((blk, D_HID), lambda i: (i, 0)),
      out_shape=jax.ShapeDtypeStruct((N, D_HID), jnp.float32),
  )(p, deg_col, z)


def _layer2_body(q_ref, dg_ref, h_ref, wl_ref, wr_ref, b2_ref, o_ref):
  inv = 1.0 / jnp.maximum(dg_ref[...], 1.0)
  agg = q_ref[...] * inv
  o_ref[...] = (jnp.dot(agg, wl_ref[...], preferred_element_type=jnp.float32)
                + jnp.dot(h_ref[...], wr_ref[...],
                          preferred_element_type=jnp.float32)
                + b2_ref[...])


def _layer2(q, deg_col, h, W2_l, W2_r, b2):
  blk = 400
  grid = N // blk
  return pl.pallas_call(
      _layer2_body,
      grid=(grid,),
      in_specs=[
          pl.BlockSpec((blk, D_HID), lambda i: (i, 0)),
          pl.BlockSpec((blk, 1), lambda i: (i, 0)),
          pl.BlockSpec((blk, D_HID), lambda i: (i, 0)),
          pl.BlockSpec((D_HID, D_OUT), lambda i: (0, 0)),
          pl.BlockSpec((D_HID, D_OUT), lambda i: (0, 0)),
          pl.BlockSpec((1, D_OUT), lambda i: (0, 0)),
      ],
      out_specs=pl.BlockSpec((blk, D_OUT), lambda i: (i, 0)),
      out_shape=jax.ShapeDtypeStruct((N, D_OUT), jnp.float32),
  )(q, deg_col, h, W2_l, W2_r, b2.reshape(1, D_OUT))


def kernel(x, edge_index, W1_l, b1, W1_r, W2_l, b2, W2_r):
  src = edge_index[0].astype(jnp.int32)
  dst = edge_index[1].astype(jnp.int32)
  pad = EP - E
  pad_src = jnp.zeros((pad,), jnp.int32)
  pad_dst = N + (jnp.arange(pad, dtype=jnp.int32) % (NP - N))
  srcp = jnp.concatenate([src, pad_src])
  dstp = jnp.concatenate([dst, pad_dst])
  ea = 16 * TKA * B
  idxa = jnp.concatenate([srcp[:ea].reshape(16, TKA, 1, B),
                          dstp[:ea].reshape(16, TKA, 1, B)], axis=2)
  idxb = jnp.concatenate([srcp[ea:].reshape(16, TKB, 1, B),
                          dstp[ea:].reshape(16, TKB, 1, B)], axis=2)

  y1, z1 = _layer1_proj(x, W1_l, W1_r, b1)
  p1, deg = _seg_sum_deg(y1, idxa, idxb)
  deg_col = deg.reshape(NP, 1)
  h = _combine1(p1, deg_col, z1)
  p2 = jax.tree.leaves(_seg_sum(h, idxa, idxb))[0]
  return _layer2(p2, deg_col, h, W2_l, W2_r, b2)


# speedup vs baseline: 1.0832x; 1.0832x over previous
import jax
import jax.numpy as jnp
from jax import lax
from jax.experimental import pallas as pl
from jax.experimental.pallas import tpu as pltpu
from jax.experimental.pallas import tpu_sc as plsc

N = 10000
E = 160000
D_IN = 256
D_HID = 128
D_OUT = 256

NP = 10240
B = 128
TKA = 64
TKB = 16
TK = TKA + TKB
EP = 16 * TK * B
RPT = NP // 16


def _make_seg_sum(tk_rows: int, with_deg: bool):
  mesh = plsc.VectorSubcoreMesh(core_axis_name="c", subcore_axis_name="s")
  out_type = [jax.ShapeDtypeStruct((NP, D_HID), jnp.float32)]
  if with_deg:
    out_type.append(jax.ShapeDtypeStruct((NP,), jnp.float32))
  scratch = [
      pltpu.VMEM((4, 2, B), jnp.int32),
      pltpu.VMEM((B, D_HID), jnp.float32),
      pltpu.VMEM((B, D_HID), jnp.float32),
      pltpu.VMEM((B,), jnp.float32),
      pltpu.VMEM((B,), jnp.float32),
      pltpu.VMEM_SHARED((NP, D_HID), jnp.float32),
  ]
  if with_deg:
    scratch.append(pltpu.VMEM_SHARED((NP,), jnp.float32))
  scratch += [pltpu.SemaphoreType.DMA] * 6

  def body(y_hbm, *rest):
    if with_deg:
      (idxa_hbm, idxb_hbm, out_rows, out_deg, idx_v, rows_v0, rows_v1,
       ones_v, zeros_v, acc_sh, deg_sh, si0, si1, si2, si3, sem0, sem1) = rest
    else:
      (idxa_hbm, out_rows, idx_v, rows_v0, rows_v1, ones_v, zeros_v,
       acc_sh, si0, si1, si2, si3, sem0, sem1) = rest
      idxb_hbm = None
      out_deg = None
      deg_sh = None
    sis = (si0, si1, si2, si3)
    cid = lax.axis_index("c")
    sid = lax.axis_index("s")
    base = sid * RPT

    one16 = jnp.ones((16,), jnp.float32)
    zero16 = jnp.zeros((16,), jnp.float32)
    for jj in range(B // 16):
      ones_v[pl.ds(jj * 16, 16)] = one16
      zeros_v[pl.ds(jj * 16, 16)] = zero16

    def zero_acc():
      rows_v = rows_v0

      def zero_rows(i, carry):
        for jj in range(D_HID // 16):
          rows_v[i, pl.ds(jj * 16, 16)] = zero16
        return carry
      lax.fori_loop(0, B, zero_rows, 0)
      for bb in range(RPT // B):
        pltpu.sync_copy(rows_v, acc_sh.at[pl.ds(base + bb * B, B)])

    def row_pipeline(idx_hbm, tk):
      def fetch_idx(j, slot):
        pltpu.async_copy(idx_hbm.at[sid, j], idx_v.at[slot], sis[slot])

      for q in range(min(4, tk)):
        fetch_idx(q, q)
      pltpu.make_async_copy(idx_hbm.at[sid, 0], idx_v.at[0], si0).wait()
      pltpu.async_copy(y_hbm.at[idx_v.at[0, 0]], rows_v0, sem0)
      pltpu.make_async_copy(idx_hbm.at[sid, 1], idx_v.at[1], si1).wait()
      pltpu.async_copy(y_hbm.at[idx_v.at[1, 0]], rows_v1, sem1)

      def halfstep(j, slot, buf, sem):
        nslot = (slot + 2) % 4
        pltpu.make_async_copy(y_hbm.at[idx_v.at[slot, 0]], buf, sem).wait()
        pltpu.sync_copy(buf, acc_sh.at[idx_v.at[slot, 1]], add=True)

        @pl.when(j + 4 < tk)
        def _():
          fetch_idx(j + 4, slot)

        @pl.when(j + 2 < tk)
        def _():
          pltpu.make_async_copy(idx_hbm.at[sid, j + 2], idx_v.at[nslot],
                                sis[nslot]).wait()
          pltpu.async_copy(y_hbm.at[idx_v.at[nslot, 0]], buf, sem)

      def step(tt, carry):
        j = 4 * tt
        halfstep(j, 0, rows_v0, sem0)
        halfstep(j + 1, 1, rows_v1, sem1)
        halfstep(j + 2, 2, rows_v0, sem0)
        halfstep(j + 3, 3, rows_v1, sem1)
        return carry
      lax.fori_loop(0, tk // 4, step, 0)

    def deg_pipeline(idx_hbm, tk):
      def fetch_idx(j, slot):
        pltpu.async_copy(idx_hbm.at[sid, j], idx_v.at[slot], sis[slot])

      for q in range(min(4, tk)):
        fetch_idx(q, q)

      def dhalf(j, slot):
        pltpu.make_async_copy(idx_hbm.at[sid, j], idx_v.at[slot],
                              sis[slot]).wait()
        pltpu.sync_copy(ones_v, deg_sh.at[idx_v.at[slot, 1]], add=True)

        @pl.when(j + 4 < tk)
        def _():
          fetch_idx(j + 4, slot)

      def dstep(tt, carry):
        j = 4 * tt
        dhalf(j, 0)
        dhalf(j + 1, 1)
        dhalf(j + 2, 2)
        dhalf(j + 3, 3)
        return carry
      lax.fori_loop(0, tk // 4, dstep, 0)

    @pl.when(cid == 0)
    def _():
      zero_acc()
      plsc.subcore_barrier()
      row_pipeline(idxa_hbm, tk_rows)
      plsc.subcore_barrier()
      pltpu.sync_copy(acc_sh.at[pl.ds(base, RPT)],
                      out_rows.at[pl.ds(base, RPT)])

    if with_deg:
      @pl.when(cid == 1)
      def _():
        for bb in range(RPT // B):
          pltpu.sync_copy(zeros_v, deg_sh.at[pl.ds(base + bb * B, B)])
        plsc.subcore_barrier()
        deg_pipeline(idxa_hbm, TKA)
        deg_pipeline(idxb_hbm, TKB)
        plsc.subcore_barrier()
        pltpu.sync_copy(deg_sh.at[pl.ds(base, RPT)],
                        out_deg.at[pl.ds(base, RPT)])

  return pl.kernel(body, mesh=mesh, out_type=out_type, scratch_types=scratch)


_seg_sum_deg_a = _make_seg_sum(TKA, with_deg=True)
_seg_sum_a = _make_seg_sum(TKA, with_deg=False)
_seg_sum_b = _make_seg_sum(TKB, with_deg=False)


def _proj_body(x_ref, wl_ref, wr_ref, b1_ref, y_ref, z_ref):
  x = x_ref[...]
  y_ref[...] = jnp.dot(x, wl_ref[...], preferred_element_type=jnp.float32)
  z_ref[...] = (jnp.dot(x, wr_ref[...], preferred_element_type=jnp.float32)
                + b1_ref[...])


def _layer1_proj(x, W1_l, W1_r, b1):
  blk = 400
  grid = N // blk
  return pl.pallas_call(
      _proj_body,
      grid=(grid,),
      in_specs=[
          pl.BlockSpec((blk, D_IN), lambda i: (i, 0)),
          pl.BlockSpec((D_IN, D_HID), lambda i: (0, 0)),
          pl.BlockSpec((D_IN, D_HID), lambda i: (0, 0)),
          pl.BlockSpec((1, D_HID), lambda i: (0, 0)),
      ],
      out_specs=[
          pl.BlockSpec((blk, D_HID), lambda i: (i, 0)),
          pl.BlockSpec((blk, D_HID), lambda i: (i, 0)),
      ],
      out_shape=[
          jax.ShapeDtypeStruct((N, D_HID), jnp.float32),
          jax.ShapeDtypeStruct((N, D_HID), jnp.float32),
      ],
  )(x, W1_l, W1_r, b1.reshape(1, D_HID))


def _combine1_body(pa_ref, pb_ref, dg_ref, z_ref, h_ref):
  inv = 1.0 / jnp.maximum(dg_ref[...], 1.0)
  h_ref[...] = jnp.maximum((pa_ref[...] + pb_ref[...]) * inv + z_ref[...], 0.0)


def _combine1(pa, pb, deg_col, z):
  blk = 400
  grid = N // blk
  return pl.pallas_call(
      _combine1_body,
      grid=(grid,),
      in_specs=[
          pl.BlockSpec((blk, D_HID), lambda i: (i, 0)),
          pl.BlockSpec((blk, D_HID), lambda i: (i, 0)),
          pl.BlockSpec((blk, 1), lambda i: (i, 0)),
          pl.BlockSpec((blk, D_HID), lambda i: (i, 0)),
      ],
      out_specs=pl.BlockSpec((blk, D_HID), lambda i: (i, 0)),
      out_shape=jax.ShapeDtypeStruct((N, D_HID), jnp.float32),
  )(pa, pb, deg_col, z)


def _layer2_body(qa_ref, qb_ref, dg_ref, h_ref, wl_ref, wr_ref, b2_ref,
                 o_ref):
  inv = 1.0 / jnp.maximum(dg_ref[...], 1.0)
  agg = (qa_ref[...] + qb_ref[...]) * inv
  o_ref[...] = (jnp.dot(agg, wl_ref[...], preferred_element_type=jnp.float32)
                + jnp.dot(h_ref[...], wr_ref[...],
                          preferred_element_type=jnp.float32)
                + b2_ref[...])


def _layer2(qa, qb, deg_col, h, W2_l, W2_r, b2):
  blk = 400
  grid = N // blk
  return pl.pallas_call(
      _layer2_body,
      grid=(grid,),
      in_specs=[
          pl.BlockSpec((blk, D_HID), lambda i: (i, 0)),
          pl.BlockSpec((blk, D_HID), lambda i: (i, 0)),
          pl.BlockSpec((blk, 1), lambda i: (i, 0)),
          pl.BlockSpec((blk, D_HID), lambda i: (i, 0)),
          pl.BlockSpec((D_HID, D_OUT), lambda i: (0, 0)),
          pl.BlockSpec((D_HID, D_OUT), lambda i: (0, 0)),
          pl.BlockSpec((1, D_OUT), lambda i: (0, 0)),
      ],
      out_specs=pl.BlockSpec((blk, D_OUT), lambda i: (i, 0)),
      out_shape=jax.ShapeDtypeStruct((N, D_OUT), jnp.float32),
  )(qa, qb, deg_col, h, W2_l, W2_r, b2.reshape(1, D_OUT))


def kernel(x, edge_index, W1_l, b1, W1_r, W2_l, b2, W2_r):
  src = edge_index[0].astype(jnp.int32)
  dst = edge_index[1].astype(jnp.int32)
  pad = EP - E
  pad_src = jnp.zeros((pad,), jnp.int32)
  pad_dst = N + (jnp.arange(pad, dtype=jnp.int32) % (NP - N))
  srcp = jnp.concatenate([src, pad_src])
  dstp = jnp.concatenate([dst, pad_dst])
  ea = 16 * TKA * B
  idxa = jnp.concatenate([srcp[:ea].reshape(16, TKA, 1, B),
                          dstp[:ea].reshape(16, TKA, 1, B)], axis=2)
  idxb = jnp.concatenate([srcp[ea:].reshape(16, TKB, 1, B),
                          dstp[ea:].reshape(16, TKB, 1, B)], axis=2)

  y1, z1 = _layer1_proj(x, W1_l, W1_r, b1)
  pa1, deg = _seg_sum_deg_a(y1, idxa, idxb)
  pb1 = jax.tree.leaves(_seg_sum_b(y1, idxb))[0]
  deg_col = deg.reshape(NP, 1)
  h = _combine1(pa1, pb1, deg_col, z1)
  pa2 = jax.tree.leaves(_seg_sum_a(h, idxa))[0]
  pb2 = jax.tree.leaves(_seg_sum_b(h, idxb))[0]
  return _layer2(pa2, pb2, deg_col, h, W2_l, W2_r, b2)


# speedup vs baseline: 1.3798x; 1.2738x over previous
import jax
import jax.numpy as jnp
from jax import lax
from jax.experimental import pallas as pl
from jax.experimental.pallas import tpu as pltpu
from jax.experimental.pallas import tpu_sc as plsc

N = 10000
E = 160000
D_IN = 256
D_HID = 128
D_OUT = 256

NP = 10240
B = 128
TKA = 64
TKB = 16
TK = TKA + TKB
EP = 16 * TK * B
RPT = NP // 16


def _make_seg_sum(with_deg: bool):
  mesh = plsc.VectorSubcoreMesh(core_axis_name="c", subcore_axis_name="s")
  out_type = [jax.ShapeDtypeStruct((2, NP, D_HID), jnp.float32)]
  if with_deg:
    out_type.append(jax.ShapeDtypeStruct((NP,), jnp.float32))
  scratch = [
      pltpu.VMEM((4, 2, B), jnp.int32),
      pltpu.VMEM((B, D_HID), jnp.float32),
      pltpu.VMEM((B, D_HID), jnp.float32),
      pltpu.VMEM((B,), jnp.float32),
      pltpu.VMEM((B,), jnp.float32),
      pltpu.VMEM_SHARED((NP, D_HID), jnp.float32),
  ]
  if with_deg:
    scratch.append(pltpu.VMEM_SHARED((NP,), jnp.float32))
  scratch += [pltpu.SemaphoreType.DMA] * 6

  def body(y_hbm, idxa_hbm, idxb_hbm, *rest):
    if with_deg:
      (out_rows, out_deg, idx_v, rows_v0, rows_v1,
       ones_v, zeros_v, acc_sh, deg_sh, si0, si1, si2, si3, sem0, sem1) = rest
    else:
      (out_rows, idx_v, rows_v0, rows_v1, ones_v, zeros_v,
       acc_sh, si0, si1, si2, si3, sem0, sem1) = rest
      out_deg = None
      deg_sh = None
    sis = (si0, si1, si2, si3)
    cid = lax.axis_index("c")
    sid = lax.axis_index("s")
    base = sid * RPT

    one16 = jnp.ones((16,), jnp.float32)
    zero16 = jnp.zeros((16,), jnp.float32)
    for jj in range(B // 16):
      ones_v[pl.ds(jj * 16, 16)] = one16
      zeros_v[pl.ds(jj * 16, 16)] = zero16

    def zero_acc():
      rows_v = rows_v0

      def zero_rows(i, carry):
        for jj in range(D_HID // 16):
          rows_v[i, pl.ds(jj * 16, 16)] = zero16
        return carry
      lax.fori_loop(0, B, zero_rows, 0)
      for bb in range(RPT // B):
        pltpu.sync_copy(rows_v, acc_sh.at[pl.ds(base + bb * B, B)])

    def row_pipeline(idx_hbm, tk):
      def fetch_idx(j, slot):
        pltpu.async_copy(idx_hbm.at[sid, j], idx_v.at[slot], sis[slot])

      for q in range(min(4, tk)):
        fetch_idx(q, q)
      pltpu.make_async_copy(idx_hbm.at[sid, 0], idx_v.at[0], si0).wait()
      pltpu.async_copy(y_hbm.at[idx_v.at[0, 0]], rows_v0, sem0)
      pltpu.make_async_copy(idx_hbm.at[sid, 1], idx_v.at[1], si1).wait()
      pltpu.async_copy(y_hbm.at[idx_v.at[1, 0]], rows_v1, sem1)

      def halfstep(j, slot, buf, sem):
        nslot = (slot + 2) % 4
        pltpu.make_async_copy(y_hbm.at[idx_v.at[slot, 0]], buf, sem).wait()
        pltpu.sync_copy(buf, acc_sh.at[idx_v.at[slot, 1]], add=True)

        @pl.when(j + 4 < tk)
        def _():
          fetch_idx(j + 4, slot)

        @pl.when(j + 2 < tk)
        def _():
          pltpu.make_async_copy(idx_hbm.at[sid, j + 2], idx_v.at[nslot],
                                sis[nslot]).wait()
          pltpu.async_copy(y_hbm.at[idx_v.at[nslot, 0]], buf, sem)

      def step(tt, carry):
        j = 4 * tt
        halfstep(j, 0, rows_v0, sem0)
        halfstep(j + 1, 1, rows_v1, sem1)
        halfstep(j + 2, 2, rows_v0, sem0)
        halfstep(j + 3, 3, rows_v1, sem1)
        return carry
      lax.fori_loop(0, tk // 4, step, 0)

    def deg_pipeline(idx_hbm, tk):
      def fetch_idx(j, slot):
        pltpu.async_copy(idx_hbm.at[sid, j], idx_v.at[slot], sis[slot])

      for q in range(min(4, tk)):
        fetch_idx(q, q)

      def dhalf(j, slot):
        pltpu.make_async_copy(idx_hbm.at[sid, j], idx_v.at[slot],
                              sis[slot]).wait()
        pltpu.sync_copy(ones_v, deg_sh.at[idx_v.at[slot, 1]], add=True)

        @pl.when(j + 4 < tk)
        def _():
          fetch_idx(j + 4, slot)

      def dstep(tt, carry):
        j = 4 * tt
        dhalf(j, 0)
        dhalf(j + 1, 1)
        dhalf(j + 2, 2)
        dhalf(j + 3, 3)
        return carry
      lax.fori_loop(0, tk // 4, dstep, 0)

    @pl.when(cid == 0)
    def _():
      zero_acc()
      plsc.subcore_barrier()
      row_pipeline(idxa_hbm, TKA)
      plsc.subcore_barrier()
      pltpu.sync_copy(acc_sh.at[pl.ds(base, RPT)],
                      out_rows.at[0, pl.ds(base, RPT)])

    @pl.when(cid == 1)
    def _():
      zero_acc()
      if with_deg:
        for bb in range(RPT // B):
          pltpu.sync_copy(zeros_v, deg_sh.at[pl.ds(base + bb * B, B)])
      plsc.subcore_barrier()
      row_pipeline(idxb_hbm, TKB)
      if with_deg:
        deg_pipeline(idxa_hbm, TKA)
        deg_pipeline(idxb_hbm, TKB)
      plsc.subcore_barrier()
      pltpu.sync_copy(acc_sh.at[pl.ds(base, RPT)],
                      out_rows.at[1, pl.ds(base, RPT)])
      if with_deg:
        pltpu.sync_copy(deg_sh.at[pl.ds(base, RPT)],
                        out_deg.at[pl.ds(base, RPT)])

  return pl.kernel(body, mesh=mesh, out_type=out_type, scratch_types=scratch)


_seg_sum_deg = _make_seg_sum(with_deg=True)
_seg_sum = _make_seg_sum(with_deg=False)


def _proj_body(x_ref, wl_ref, wr_ref, b1_ref, y_ref, z_ref):
  x = x_ref[...]
  y_ref[...] = jnp.dot(x, wl_ref[...], preferred_element_type=jnp.float32)
  z_ref[...] = (jnp.dot(x, wr_ref[...], preferred_element_type=jnp.float32)
                + b1_ref[...])


def _layer1_proj(x, W1_l, W1_r, b1):
  blk = 400
  grid = N // blk
  return pl.pallas_call(
      _proj_body,
      grid=(grid,),
      in_specs=[
          pl.BlockSpec((blk, D_IN), lambda i: (i, 0)),
          pl.BlockSpec((D_IN, D_HID), lambda i: (0, 0)),
          pl.BlockSpec((D_IN, D_HID), lambda i: (0, 0)),
          pl.BlockSpec((1, D_HID), lambda i: (0, 0)),
      ],
      out_specs=[
          pl.BlockSpec((blk, D_HID), lambda i: (i, 0)),
          pl.BlockSpec((blk, D_HID), lambda i: (i, 0)),
      ],
      out_shape=[
          jax.ShapeDtypeStruct((N, D_HID), jnp.float32),
          jax.ShapeDtypeStruct((N, D_HID), jnp.float32),
      ],
  )(x, W1_l, W1_r, b1.reshape(1, D_HID))


def _combine1_body(p_ref, dg_ref, z_ref, h_ref):
  inv = 1.0 / jnp.maximum(dg_ref[...], 1.0)
  h_ref[...] = jnp.maximum((p_ref[0] + p_ref[1]) * inv + z_ref[...], 0.0)


def _combine1(p, deg_col, z):
  blk = 400
  grid = N // blk
  return pl.pallas_call(
      _combine1_body,
      grid=(grid,),
      in_specs=[
          pl.BlockSpec((2, blk, D_HID), lambda i: (0, i, 0)),
          pl.BlockSpec((blk, 1), lambda i: (i, 0)),
          pl.BlockSpec((blk, D_HID), lambda i: (i, 0)),
      ],
      out_specs=pl.BlockSpec((blk, D_HID), lambda i: (i, 0)),
      out_shape=jax.ShapeDtypeStruct((N, D_HID), jnp.float32),
  )(p, deg_col, z)


def _layer2_body(q_ref, dg_ref, h_ref, wl_ref, wr_ref, b2_ref, o_ref):
  inv = 1.0 / jnp.maximum(dg_ref[...], 1.0)
  agg = (q_ref[0] + q_ref[1]) * inv
  o_ref[...] = (jnp.dot(agg, wl_ref[...], preferred_element_type=jnp.float32)
                + jnp.dot(h_ref[...], wr_ref[...],
                          preferred_element_type=jnp.float32)
                + b2_ref[...])


def _layer2(q, deg_col, h, W2_l, W2_r, b2):
  blk = 400
  grid = N // blk
  return pl.pallas_call(
      _layer2_body,
      grid=(grid,),
      in_specs=[
          pl.BlockSpec((2, blk, D_HID), lambda i: (0, i, 0)),
          pl.BlockSpec((blk, 1), lambda i: (i, 0)),
          pl.BlockSpec((blk, D_HID), lambda i: (i, 0)),
          pl.BlockSpec((D_HID, D_OUT), lambda i: (0, 0)),
          pl.BlockSpec((D_HID, D_OUT), lambda i: (0, 0)),
          pl.BlockSpec((1, D_OUT), lambda i: (0, 0)),
      ],
      out_specs=pl.BlockSpec((blk, D_OUT), lambda i: (i, 0)),
      out_shape=jax.ShapeDtypeStruct((N, D_OUT), jnp.float32),
  )(q, deg_col, h, W2_l, W2_r, b2.reshape(1, D_OUT))


def kernel(x, edge_index, W1_l, b1, W1_r, W2_l, b2, W2_r):
  src = edge_index[0].astype(jnp.int32)
  dst = edge_index[1].astype(jnp.int32)
  pad = EP - E
  pad_src = jnp.zeros((pad,), jnp.int32)
  pad_dst = N + (jnp.arange(pad, dtype=jnp.int32) % (NP - N))
  srcp = jnp.concatenate([src, pad_src])
  dstp = jnp.concatenate([dst, pad_dst])
  ea = 16 * TKA * B
  idxa = jnp.concatenate([srcp[:ea].reshape(16, TKA, 1, B),
                          dstp[:ea].reshape(16, TKA, 1, B)], axis=2)
  idxb = jnp.concatenate([srcp[ea:].reshape(16, TKB, 1, B),
                          dstp[ea:].reshape(16, TKB, 1, B)], axis=2)

  y1, z1 = _layer1_proj(x, W1_l, W1_r, b1)
  p1, deg = _seg_sum_deg(y1, idxa, idxb)
  deg_col = deg.reshape(NP, 1)
  h = _combine1(p1, deg_col, z1)
  p2 = jax.tree.leaves(_seg_sum(h, idxa, idxb))[0]
  return _layer2(p2, deg_col, h, W2_l, W2_r, b2)
